# baseline (speedup 1.0000x reference)
import functools

import jax
import jax.numpy as jnp
from jax import lax
from jax.experimental import pallas as pl
from jax.experimental.pallas import tpu as pltpu
from jax.experimental.pallas import tpu_sc as plsc

F32 = jnp.float32
I32 = jnp.int32

SDIM = 64
EDGE_DIM = 16
NUM_GRAPHS = 128
BLK = 128
BINSZ = NUM_GRAPHS + 16


def _node_stage_body(x_ref, emb_ref, w1a_ref, w1b_ref, wg_ref,
                     pa_ref, pb_ref, a_ref):
    rows = x_ref.shape[0]
    iot = lax.broadcasted_iota(I32, (1, 64), 1)
    s = jnp.zeros((rows, SDIM), F32)
    for i in range(9):
        oh = (x_ref[:, i:i + 1] == iot).astype(F32)
        s = s + jnp.dot(oh, emb_ref[pl.ds(i * 64, 64), :],
                        preferred_element_type=F32, precision=lax.Precision.HIGHEST)
    pa_ref[...] = jnp.dot(s, w1a_ref[...], preferred_element_type=F32, precision=lax.Precision.HIGHEST)
    pb_ref[...] = jnp.dot(s, w1b_ref[...], preferred_element_type=F32, precision=lax.Precision.HIGHEST)
    a_ref[...] = jnp.dot(s, wg_ref[...], preferred_element_type=F32, precision=lax.Precision.HIGHEST)


def _node_stage(x, emb, w1a, w1b, wg):
    n = x.shape[0]
    r = 2000
    grid = (n // r,)
    return pl.pallas_call(
        _node_stage_body,
        grid=grid,
        in_specs=[
            pl.BlockSpec((r, 9), lambda i: (i, 0)),
            pl.BlockSpec((576, 64), lambda i: (0, 0)),
            pl.BlockSpec((64, 64), lambda i: (0, 0)),
            pl.BlockSpec((64, 64), lambda i: (0, 0)),
            pl.BlockSpec((64, 1), lambda i: (0, 0)),
        ],
        out_specs=[
            pl.BlockSpec((r, 64), lambda i: (i, 0)),
            pl.BlockSpec((r, 64), lambda i: (i, 0)),
            pl.BlockSpec((r, 1), lambda i: (i, 0)),
        ],
        out_shape=[
            jax.ShapeDtypeStruct((n, 64), F32),
            jax.ShapeDtypeStruct((n, 64), F32),
            jax.ShapeDtypeStruct((n, 1), F32),
        ],
    )(x, emb, w1a, w1b, wg)


def _bond_stage_body(be_ref, w1c_ref, b1_ref, out_ref):
    r = lax.broadcasted_iota(I32, (512, 1), 0)
    i8 = lax.broadcasted_iota(I32, (1, 8), 1)
    eb = jnp.zeros((512, EDGE_DIM), F32)
    for i, div in enumerate((64, 8, 1)):
        oh = ((r // div) % 8 == i8).astype(F32)
        eb = eb + jnp.dot(oh, be_ref[i], preferred_element_type=F32, precision=lax.Precision.HIGHEST)
    out_ref[...] = jnp.dot(eb, w1c_ref[...], preferred_element_type=F32, precision=lax.Precision.HIGHEST) \
        + b1_ref[...]


def _bond_stage(bond_emb, w1c, b1):
    return pl.pallas_call(
        _bond_stage_body,
        out_shape=jax.ShapeDtypeStruct((512, 64), F32),
    )(bond_emb, w1c, b1)


_MLP_BLOCKS = 125


def _mlp_body(u_ref, bf_ref, w2_ref, b2_ref, out_ref):
    @pl.when(pl.program_id(0) == 0)
    def _():
        out_ref[...] = jnp.zeros_like(out_ref)

    u = u_ref[...]
    s = u / (1.0 + jnp.exp(-u))
    y = jnp.dot(s, w2_ref[...], preferred_element_type=F32,
                precision=lax.Precision.HIGHEST) + b2_ref[0, 0]
    base = pl.program_id(0) * _MLP_BLOCKS
    giota = lax.broadcasted_iota(I32, (NUM_GRAPHS, 1), 0).astype(F32)
    acc = jnp.zeros((NUM_GRAPHS, 1), F32)
    for b in range(_MLP_BLOCKS):
        row = bf_ref[pl.ds(base + b, 1), :]
        oht = (giota == row).astype(F32)
        acc = acc + jnp.dot(oht, y[128 * b:128 * (b + 1), :],
                            preferred_element_type=F32,
                            precision=lax.Precision.HIGHEST)
    out_ref[...] += acc


def _mlp_stage(u, bate2d, w2, b2):
    rows = u.shape[0]
    r = 128 * _MLP_BLOCKS
    grid = (rows // r,)
    return pl.pallas_call(
        _mlp_body,
        grid=grid,
        in_specs=[
            pl.BlockSpec((r, 64), lambda i: (i, 0)),
            pl.BlockSpec(bate2d.shape, lambda i: (0, 0)),
            pl.BlockSpec((64, 1), lambda i: (0, 0)),
            pl.BlockSpec((1, 1), lambda i: (0, 0)),
        ],
        out_specs=pl.BlockSpec((NUM_GRAPHS, 1), lambda i: (0, 0)),
        out_shape=jax.ShapeDtypeStruct((NUM_GRAPHS, 1), F32),
    )(u, bate2d, w2, b2)


def _reduce_body(p_ref, t_ref, o_ref):
    x = p_ref[...]
    o_ref[...] = jnp.sum(x[:, :NUM_GRAPHS], axis=0, keepdims=True) + t_ref[...]


def _reduce_stage(partials, tc_energy):
    return pl.pallas_call(
        _reduce_body,
        out_shape=jax.ShapeDtypeStruct((1, NUM_GRAPHS), F32),
    )(partials, tc_energy)


def _make_sc_main(n_edges, nc, ns):
    nw = nc * ns
    nblk = n_edges // BLK
    nbt = (nblk + nw - 1) // nw
    npair = (nbt + 1) // 2

    def body(pa_hbm, pb_hbm, bond_hbm, px_hbm, py_hbm, pz_hbm, bat_hbm,
             batf_hbm, av_hbm, srcl_hbm, dstl_hbm, bidx_hbm, srcg_hbm,
             dstg_hbm, consts_hbm, out_hbm, u_hbm, bate_hbm,
             ab0, bb0, sib0, dib0, bib0,
             c10, c20, c30, c40, c50, c60, c70, c80, bat0,
             ab1, bb1, sib1, dib1, bib1,
             c11, c21, c31, c41, c51, c61, c71, c81, bat1,
             bondv, cv, bins, sem_i0, sem_g0, sem_i1, sem_g1,
             sem_s0, sem_s1):
        wid = lax.axis_index("s") * nc + lax.axis_index("c")
        pltpu.sync_copy(bond_hbm, bondv)
        pltpu.sync_copy(consts_hbm, cv)
        z16 = jnp.zeros((16,), F32)
        for k in range(BINSZ // 16):
            bins[pl.ds(k * 16, 16)] = z16
        w2 = [cv[pl.ds(c * 16, 16)] for c in range(4)]
        w1d = [cv[pl.ds(64 + c * 16, 16)] for c in range(4)]
        b2s = cv[pl.ds(128, 16)][0]
        iota16 = lax.iota(I32, 16)

        sets = [
            dict(ab=ab0, bb=bb0, sib=sib0, dib=dib0, bib=bib0,
                 c=[c10, c20, c30, c40, c50, c60, c70, c80], bat=bat0,
                 si=sem_i0, sg=sem_g0, ss=sem_s0),
            dict(ab=ab1, bb=bb1, sib=sib1, dib=dib1, bib=bib1,
                 c=[c11, c21, c31, c41, c51, c61, c71, c81], bat=bat1,
                 si=sem_i1, sg=sem_g1, ss=sem_s1),
        ]

        def nr_sqrt(d2):
            seed = jnp.zeros((16,), F32)
            for e in range(16):
                ri = lax.bitcast_convert_type(d2[e], I32)
                ri = 0x5F3759DF - lax.shift_right_logical(ri, 1)
                seed = jnp.where(iota16 == e,
                                 lax.bitcast_convert_type(ri, F32), seed)
            r = seed
            for _ in range(3):
                r = r * (1.5 - 0.5 * d2 * r * r)
            return d2 * r

        def bin_add(bt_e, val):
            w = bins[pl.ds(bt_e, 16)]
            bins[pl.ds(bt_e, 16)] = w + jnp.where(iota16 == 0, val, 0.0)

        def idx_copies_local(S, base):
            return [
                pltpu.make_async_copy(srcl_hbm.at[pl.ds(base, BLK)],
                                      S["sib"], S["si"]),
                pltpu.make_async_copy(dstl_hbm.at[pl.ds(base, BLK)],
                                      S["dib"], S["si"]),
                pltpu.make_async_copy(bidx_hbm.at[pl.ds(base, BLK)],
                                      S["bib"], S["si"]),
            ]

        def gather_copies_local(S):
            c = S["c"]
            return [
                pltpu.make_async_copy(pa_hbm.at[S["sib"]], S["ab"], S["sg"]),
                pltpu.make_async_copy(pb_hbm.at[S["dib"]], S["bb"], S["sg"]),
                pltpu.make_async_copy(px_hbm.at[S["sib"]], c[0], S["sg"]),
                pltpu.make_async_copy(py_hbm.at[S["sib"]], c[1], S["sg"]),
                pltpu.make_async_copy(pz_hbm.at[S["sib"]], c[2], S["sg"]),
                pltpu.make_async_copy(px_hbm.at[S["dib"]], c[3], S["sg"]),
                pltpu.make_async_copy(py_hbm.at[S["dib"]], c[4], S["sg"]),
                pltpu.make_async_copy(pz_hbm.at[S["dib"]], c[5], S["sg"]),
                pltpu.make_async_copy(batf_hbm.at[S["dib"]], c[6], S["sg"]),
            ]

        def store_copies_local(S, base):
            return [
                pltpu.make_async_copy(S["ab"], u_hbm.at[pl.ds(base, BLK)],
                                      S["ss"]),
                pltpu.make_async_copy(S["c"][6],
                                      bate_hbm.at[pl.ds(base, BLK)], S["ss"]),
            ]

        def idx_copies_global(S, base):
            return [
                pltpu.make_async_copy(srcg_hbm.at[pl.ds(base, BLK)],
                                      S["sib"], S["si"]),
                pltpu.make_async_copy(dstg_hbm.at[pl.ds(base, BLK)],
                                      S["dib"], S["si"]),
            ]

        def gather_copies_global(S):
            c = S["c"]
            return [
                pltpu.make_async_copy(av_hbm.at[S["sib"]], c[6], S["sg"]),
                pltpu.make_async_copy(av_hbm.at[S["dib"]], c[7], S["sg"]),
                pltpu.make_async_copy(px_hbm.at[S["sib"]], c[0], S["sg"]),
                pltpu.make_async_copy(py_hbm.at[S["sib"]], c[1], S["sg"]),
                pltpu.make_async_copy(pz_hbm.at[S["sib"]], c[2], S["sg"]),
                pltpu.make_async_copy(px_hbm.at[S["dib"]], c[3], S["sg"]),
                pltpu.make_async_copy(py_hbm.at[S["dib"]], c[4], S["sg"]),
                pltpu.make_async_copy(pz_hbm.at[S["dib"]], c[5], S["sg"]),
                pltpu.make_async_copy(bat_hbm.at[S["dib"]], S["bat"], S["sg"]),
            ]

        def fetch_block(idx_fn, gather_fn, S, bb):
            ics = idx_fn(S, bb * BLK)
            for ic in ics:
                ic.start()
            for ic in ics:
                ic.wait()
            for gc in gather_fn(S):
                gc.start()

        def wait_gathers(gather_fn, S):
            for gc in gather_fn(S):
                gc.wait()

        def compute_local(S, bb):
            ab, bb_, bib = S["ab"], S["bb"], S["bib"]
            c = S["c"]

            def group(j, carry2):
                sl16 = pl.ds(j * 16, 16)
                dx = c[3][sl16] - c[0][sl16]
                dy = c[4][sl16] - c[1][sl16]
                dz = c[5][sl16] - c[2][sl16]
                d16 = nr_sqrt(dx * dx + dy * dy + dz * dz + 1e-12)
                bidx16 = bib[sl16]
                for e in range(16):
                    row = j * 16 + e
                    d_e = d16[e]
                    bre = bidx16[e]
                    for cc in range(4):
                        sl = pl.ds(cc * 16, 16)
                        ab[row, sl] = (ab[row, sl] + bb_[row, sl]
                                       + bondv[bre, sl] + d_e * w1d[cc])
                return carry2

            lax.fori_loop(0, BLK // 16, group, 0)
            for sc_ in store_copies_local(S, bb * BLK):
                sc_.start()

        def compute_global(S, bb):
            batb = S["bat"]
            c = S["c"]

            def group(j, carry2):
                sl16 = pl.ds(j * 16, 16)
                dx = c[3][sl16] - c[0][sl16]
                dy = c[4][sl16] - c[1][sl16]
                dz = c[5][sl16] - c[2][sl16]
                d2 = dx * dx + dy * dy + dz * dz
                eg = c[6][sl16] * c[7][sl16] / (d2 + 1.0)
                btv = batb[sl16]
                for e in range(16):
                    bin_add(btv[e], eg[e])
                return carry2

            lax.fori_loop(0, BLK // 16, group, 0)

        def run_pipelined(idx_fn, gather_fn, compute_fn, store_fn=None):
            def wait_stores(S):
                for sc_ in store_fn(S, 0):
                    sc_.wait()

            @pl.when(wid < nblk)
            def _():
                fetch_block(idx_fn, gather_fn, sets[0], wid)

            def pair(h, carry):
                for p in (0, 1):
                    g = 2 * h + p
                    bb = wid + g * nw
                    bbn = bb + nw

                    @pl.when(bbn < nblk)
                    def _(q=1 - p, bbn=bbn, p=p, h=h):
                        if store_fn is not None:
                            if p == 0:
                                @pl.when(h >= 1)
                                def _(q=q):
                                    wait_stores(sets[q])
                            else:
                                wait_stores(sets[q])
                        fetch_block(idx_fn, gather_fn, sets[q], bbn)

                    @pl.when(bb < nblk)
                    def _(p=p, bb=bb):
                        wait_gathers(gather_fn, sets[p])
                        compute_fn(sets[p], bb)
                return carry

            lax.fori_loop(0, npair, pair, 0)
            if store_fn is not None:
                @pl.when(wid < nblk)
                def _():
                    wait_stores(sets[0])

                @pl.when(wid + nw < nblk)
                def _():
                    wait_stores(sets[1])

        run_pipelined(idx_copies_local, gather_copies_local, compute_local,
                      store_copies_local)
        run_pipelined(idx_copies_global, gather_copies_global, compute_global)
        pltpu.sync_copy(bins, out_hbm.at[wid])

    buf_set = [
        pltpu.VMEM((BLK, SDIM), F32),
        pltpu.VMEM((BLK, SDIM), F32),
        pltpu.VMEM((BLK,), I32),
        pltpu.VMEM((BLK,), I32),
        pltpu.VMEM((BLK,), I32),
        pltpu.VMEM((BLK,), F32),
        pltpu.VMEM((BLK,), F32),
        pltpu.VMEM((BLK,), F32),
        pltpu.VMEM((BLK,), F32),
        pltpu.VMEM((BLK,), F32),
        pltpu.VMEM((BLK,), F32),
        pltpu.VMEM((BLK,), F32),
        pltpu.VMEM((BLK,), F32),
        pltpu.VMEM((BLK,), I32),
    ]
    return pl.kernel(
        body,
        mesh=plsc.VectorSubcoreMesh(core_axis_name="c", subcore_axis_name="s"),
        compiler_params=pltpu.CompilerParams(use_tc_tiling_on_sc=False),
        out_type=[
            jax.ShapeDtypeStruct((nw, BINSZ), F32),
            jax.ShapeDtypeStruct((n_edges, SDIM), F32),
            jax.ShapeDtypeStruct((n_edges,), F32),
        ],
        scratch_types=buf_set + buf_set + [
            pltpu.VMEM((512, 64), F32),
            pltpu.VMEM((144,), F32),
            pltpu.VMEM((BINSZ,), F32),
            pltpu.SemaphoreType.DMA,
            pltpu.SemaphoreType.DMA,
            pltpu.SemaphoreType.DMA,
            pltpu.SemaphoreType.DMA,
            pltpu.SemaphoreType.DMA,
            pltpu.SemaphoreType.DMA,
        ],
    )


def kernel(x, pos, edge_index_local, edge_index_global, edge_attr_local,
           edge_attr_global, batch, atom_emb, bond_emb, W1, b1, W2, b2, w_g):
    n = x.shape[0]
    n_edges = edge_index_local.shape[1]
    x = x.astype(I32)
    pos = pos.astype(F32)
    srcl = edge_index_local[0].astype(I32)
    dstl = edge_index_local[1].astype(I32)
    srcg = edge_index_global[0].astype(I32)
    dstg = edge_index_global[1].astype(I32)
    eal = edge_attr_local.astype(I32)
    bidx = eal[:, 0] * 64 + eal[:, 1] * 8 + eal[:, 2]
    bat = batch.astype(I32)

    emb = atom_emb.astype(F32).reshape(9 * 64, 64)
    W1 = W1.astype(F32)
    pa, pb, av = _node_stage(x, emb, W1[:64], W1[64:128],
                             w_g.astype(F32).reshape(64, 1))
    bond_all = _bond_stage(bond_emb.astype(F32), W1[128:144],
                           b1.astype(F32).reshape(1, 64))
    consts = jnp.concatenate([W2[:, 0].astype(F32), W1[144],
                              b2.astype(F32), jnp.zeros((15,), F32)])

    try:
        info = plsc.get_sparse_core_info()
        nc, ns = info.num_cores, info.num_subcores
    except Exception:
        nc, ns = 2, 16
    sc_main = _make_sc_main(n_edges, nc, ns)
    partials, u, bate = sc_main(
        pa, pb, bond_all, pos[:, 0], pos[:, 1], pos[:, 2],
        bat, bat.astype(F32), av.reshape(n), srcl, dstl, bidx, srcg, dstg,
        consts)
    tc_energy = _mlp_stage(u, bate.reshape(n_edges // 128, 128),
                           W2.astype(F32).reshape(64, 1),
                           b2.astype(F32).reshape(1, 1))
    out = _reduce_stage(partials, tc_energy.reshape(1, NUM_GRAPHS))
    return out.reshape(NUM_GRAPHS)


# speedup vs baseline: 1.3405x; 1.3405x over previous
import functools

import jax
import jax.numpy as jnp
from jax import lax
from jax.experimental import pallas as pl
from jax.experimental.pallas import tpu as pltpu
from jax.experimental.pallas import tpu_sc as plsc

F32 = jnp.float32
I32 = jnp.int32

SDIM = 64
EDGE_DIM = 16
NUM_GRAPHS = 128
BLK = 128
BINSZ = NUM_GRAPHS + 16


def _node_stage_body(x_ref, emb_ref, w1a_ref, w1b_ref, wg_ref,
                     pa_ref, pb_ref, a_ref):
    rows = x_ref.shape[0]
    iot = lax.broadcasted_iota(I32, (1, 64), 1)
    s = jnp.zeros((rows, SDIM), F32)
    for i in range(9):
        oh = (x_ref[:, i:i + 1] == iot).astype(F32)
        s = s + jnp.dot(oh, emb_ref[pl.ds(i * 64, 64), :],
                        preferred_element_type=F32, precision=lax.Precision.HIGHEST)
    pa_ref[...] = jnp.dot(s, w1a_ref[...], preferred_element_type=F32, precision=lax.Precision.HIGHEST)
    pb_ref[...] = jnp.dot(s, w1b_ref[...], preferred_element_type=F32, precision=lax.Precision.HIGHEST)
    a_ref[...] = jnp.dot(s, wg_ref[...], preferred_element_type=F32, precision=lax.Precision.HIGHEST)


def _node_stage(x, emb, w1a, w1b, wg):
    n = x.shape[0]
    r = 2000
    grid = (n // r,)
    return pl.pallas_call(
        _node_stage_body,
        grid=grid,
        in_specs=[
            pl.BlockSpec((r, 9), lambda i: (i, 0)),
            pl.BlockSpec((576, 64), lambda i: (0, 0)),
            pl.BlockSpec((64, 64), lambda i: (0, 0)),
            pl.BlockSpec((64, 64), lambda i: (0, 0)),
            pl.BlockSpec((64, 1), lambda i: (0, 0)),
        ],
        out_specs=[
            pl.BlockSpec((r, 64), lambda i: (i, 0)),
            pl.BlockSpec((r, 64), lambda i: (i, 0)),
            pl.BlockSpec((r, 1), lambda i: (i, 0)),
        ],
        out_shape=[
            jax.ShapeDtypeStruct((n, 64), F32),
            jax.ShapeDtypeStruct((n, 64), F32),
            jax.ShapeDtypeStruct((n, 1), F32),
        ],
    )(x, emb, w1a, w1b, wg)


def _bond_stage_body(be_ref, w1c_ref, b1_ref, out_ref):
    r = lax.broadcasted_iota(I32, (512, 1), 0)
    i8 = lax.broadcasted_iota(I32, (1, 8), 1)
    eb = jnp.zeros((512, EDGE_DIM), F32)
    for i, div in enumerate((64, 8, 1)):
        oh = ((r // div) % 8 == i8).astype(F32)
        eb = eb + jnp.dot(oh, be_ref[i], preferred_element_type=F32, precision=lax.Precision.HIGHEST)
    out_ref[...] = jnp.dot(eb, w1c_ref[...], preferred_element_type=F32, precision=lax.Precision.HIGHEST) \
        + b1_ref[...]


def _bond_stage(bond_emb, w1c, b1):
    return pl.pallas_call(
        _bond_stage_body,
        out_shape=jax.ShapeDtypeStruct((512, 64), F32),
    )(bond_emb, w1c, b1)


_SEG = 3200
_MLP_ROWS = 16000


def _mlp_body(u_ref, bf_ref, w2_ref, b2_ref, out_ref):
    @pl.when(pl.program_id(0) == 0)
    def _():
        out_ref[...] = jnp.zeros_like(out_ref)

    u = u_ref[...]
    s = u / (1.0 + jnp.exp(-u))
    giota = lax.broadcasted_iota(I32, (NUM_GRAPHS, 1), 0).astype(F32)
    nchunk = _MLP_ROWS // _SEG
    nsub = _SEG // 128
    base = pl.program_id(0) * (_MLP_ROWS // 128)
    acc = jnp.zeros((NUM_GRAPHS, SDIM), F32)
    cnt = jnp.zeros((NUM_GRAPHS, 1), F32)
    for c in range(nchunk):
        oh = jnp.concatenate(
            [(giota == bf_ref[pl.ds(base + nsub * c + j, 1), :]).astype(F32)
             for j in range(nsub)], axis=1)
        acc = acc + jnp.dot(oh, s[_SEG * c:_SEG * (c + 1), :],
                            preferred_element_type=F32,
                            precision=lax.Precision.HIGHEST)
        cnt = cnt + jnp.sum(oh, axis=1, keepdims=True)
    out_ref[...] += jnp.dot(acc, w2_ref[...], preferred_element_type=F32,
                            precision=lax.Precision.HIGHEST) + b2_ref[0, 0] * cnt


def _mlp_stage(u, bate2d, w2, b2):
    rows = u.shape[0]
    grid = (rows // _MLP_ROWS,)
    return pl.pallas_call(
        _mlp_body,
        grid=grid,
        in_specs=[
            pl.BlockSpec((_MLP_ROWS, 64), lambda i: (i, 0)),
            pl.BlockSpec(bate2d.shape, lambda i: (0, 0)),
            pl.BlockSpec((64, 1), lambda i: (0, 0)),
            pl.BlockSpec((1, 1), lambda i: (0, 0)),
        ],
        out_specs=pl.BlockSpec((NUM_GRAPHS, 1), lambda i: (0, 0)),
        out_shape=jax.ShapeDtypeStruct((NUM_GRAPHS, 1), F32),
    )(u, bate2d, w2, b2)


def _reduce_body(p_ref, t_ref, o_ref):
    x = p_ref[...]
    o_ref[...] = jnp.sum(x[:, :NUM_GRAPHS], axis=0, keepdims=True) + t_ref[...]


def _reduce_stage(partials, tc_energy):
    return pl.pallas_call(
        _reduce_body,
        out_shape=jax.ShapeDtypeStruct((1, NUM_GRAPHS), F32),
    )(partials, tc_energy)


def _make_sc_main(n_edges, nc, ns):
    nw = nc * ns
    nblk = n_edges // BLK
    nbt = (nblk + nw - 1) // nw
    npair = (nbt + 1) // 2

    def body(pa_hbm, pb_hbm, bond_hbm, px_hbm, py_hbm, pz_hbm, bat_hbm,
             batf_hbm, av_hbm, srcl_hbm, dstl_hbm, bidx_hbm, srcg_hbm,
             dstg_hbm, consts_hbm, out_hbm, u_hbm, bate_hbm,
             ab0, bb0, sib0, dib0, bib0,
             c10, c20, c30, c40, c50, c60, c70, c80, bat0,
             ab1, bb1, sib1, dib1, bib1,
             c11, c21, c31, c41, c51, c61, c71, c81, bat1,
             bondv, cv, bins, sem_i0, sem_g0, sem_i1, sem_g1,
             sem_s0, sem_s1):
        wid = lax.axis_index("s") * nc + lax.axis_index("c")
        pltpu.sync_copy(bond_hbm, bondv)
        pltpu.sync_copy(consts_hbm, cv)
        z16 = jnp.zeros((16,), F32)
        for k in range(BINSZ // 16):
            bins[pl.ds(k * 16, 16)] = z16
        w2 = [cv[pl.ds(c * 16, 16)] for c in range(4)]
        w1d = [cv[pl.ds(64 + c * 16, 16)] for c in range(4)]
        b2s = cv[pl.ds(128, 16)][0]
        iota16 = lax.iota(I32, 16)

        sets = [
            dict(ab=ab0, bb=bb0, sib=sib0, dib=dib0, bib=bib0,
                 c=[c10, c20, c30, c40, c50, c60, c70, c80], bat=bat0,
                 si=sem_i0, sg=sem_g0, ss=sem_s0),
            dict(ab=ab1, bb=bb1, sib=sib1, dib=dib1, bib=bib1,
                 c=[c11, c21, c31, c41, c51, c61, c71, c81], bat=bat1,
                 si=sem_i1, sg=sem_g1, ss=sem_s1),
        ]

        def nr_sqrt(d2):
            seed = jnp.zeros((16,), F32)
            for e in range(16):
                ri = lax.bitcast_convert_type(d2[e], I32)
                ri = 0x5F3759DF - lax.shift_right_logical(ri, 1)
                seed = jnp.where(iota16 == e,
                                 lax.bitcast_convert_type(ri, F32), seed)
            r = seed
            for _ in range(3):
                r = r * (1.5 - 0.5 * d2 * r * r)
            return d2 * r

        def bin_add(bt_e, val):
            w = bins[pl.ds(bt_e, 16)]
            bins[pl.ds(bt_e, 16)] = w + jnp.where(iota16 == 0, val, 0.0)

        def idx_copies_local(S, base):
            return [
                pltpu.make_async_copy(srcl_hbm.at[pl.ds(base, BLK)],
                                      S["sib"], S["si"]),
                pltpu.make_async_copy(dstl_hbm.at[pl.ds(base, BLK)],
                                      S["dib"], S["si"]),
                pltpu.make_async_copy(bidx_hbm.at[pl.ds(base, BLK)],
                                      S["bib"], S["si"]),
            ]

        def gather_copies_local(S):
            c = S["c"]
            return [
                pltpu.make_async_copy(pa_hbm.at[S["sib"]], S["ab"], S["sg"]),
                pltpu.make_async_copy(pb_hbm.at[S["dib"]], S["bb"], S["sg"]),
                pltpu.make_async_copy(px_hbm.at[S["sib"]], c[0], S["sg"]),
                pltpu.make_async_copy(py_hbm.at[S["sib"]], c[1], S["sg"]),
                pltpu.make_async_copy(pz_hbm.at[S["sib"]], c[2], S["sg"]),
                pltpu.make_async_copy(px_hbm.at[S["dib"]], c[3], S["sg"]),
                pltpu.make_async_copy(py_hbm.at[S["dib"]], c[4], S["sg"]),
                pltpu.make_async_copy(pz_hbm.at[S["dib"]], c[5], S["sg"]),
                pltpu.make_async_copy(batf_hbm.at[S["dib"]], c[6], S["sg"]),
            ]

        def store_copies_local(S, base):
            return [
                pltpu.make_async_copy(S["ab"], u_hbm.at[pl.ds(base, BLK)],
                                      S["ss"]),
                pltpu.make_async_copy(S["c"][6],
                                      bate_hbm.at[pl.ds(base, BLK)], S["ss"]),
            ]

        def idx_copies_global(S, base):
            return [
                pltpu.make_async_copy(srcg_hbm.at[pl.ds(base, BLK)],
                                      S["sib"], S["si"]),
                pltpu.make_async_copy(dstg_hbm.at[pl.ds(base, BLK)],
                                      S["dib"], S["si"]),
            ]

        def gather_copies_global(S):
            c = S["c"]
            return [
                pltpu.make_async_copy(av_hbm.at[S["sib"]], c[6], S["sg"]),
                pltpu.make_async_copy(av_hbm.at[S["dib"]], c[7], S["sg"]),
                pltpu.make_async_copy(px_hbm.at[S["sib"]], c[0], S["sg"]),
                pltpu.make_async_copy(py_hbm.at[S["sib"]], c[1], S["sg"]),
                pltpu.make_async_copy(pz_hbm.at[S["sib"]], c[2], S["sg"]),
                pltpu.make_async_copy(px_hbm.at[S["dib"]], c[3], S["sg"]),
                pltpu.make_async_copy(py_hbm.at[S["dib"]], c[4], S["sg"]),
                pltpu.make_async_copy(pz_hbm.at[S["dib"]], c[5], S["sg"]),
                pltpu.make_async_copy(bat_hbm.at[S["dib"]], S["bat"], S["sg"]),
            ]

        def fetch_block(idx_fn, gather_fn, S, bb):
            ics = idx_fn(S, bb * BLK)
            for ic in ics:
                ic.start()
            for ic in ics:
                ic.wait()
            for gc in gather_fn(S):
                gc.start()

        def wait_gathers(gather_fn, S):
            for gc in gather_fn(S):
                gc.wait()

        def compute_local(S, bb):
            ab, bb_, bib = S["ab"], S["bb"], S["bib"]
            c = S["c"]

            def group(j, carry2):
                sl16 = pl.ds(j * 16, 16)
                dx = c[3][sl16] - c[0][sl16]
                dy = c[4][sl16] - c[1][sl16]
                dz = c[5][sl16] - c[2][sl16]
                d16 = nr_sqrt(dx * dx + dy * dy + dz * dz + 1e-12)
                bidx16 = bib[sl16]
                for e in range(16):
                    row = j * 16 + e
                    d_e = d16[e]
                    bre = bidx16[e]
                    for cc in range(4):
                        sl = pl.ds(cc * 16, 16)
                        ab[row, sl] = (ab[row, sl] + bb_[row, sl]
                                       + bondv[bre, sl] + d_e * w1d[cc])
                return carry2

            lax.fori_loop(0, BLK // 16, group, 0)
            for sc_ in store_copies_local(S, bb * BLK):
                sc_.start()

        def compute_global(S, bb):
            batb = S["bat"]
            c = S["c"]

            def group(j, carry2):
                sl16 = pl.ds(j * 16, 16)
                dx = c[3][sl16] - c[0][sl16]
                dy = c[4][sl16] - c[1][sl16]
                dz = c[5][sl16] - c[2][sl16]
                d2 = dx * dx + dy * dy + dz * dz
                eg = c[6][sl16] * c[7][sl16] / (d2 + 1.0)
                btv = batb[sl16]
                for e in range(16):
                    bin_add(btv[e], eg[e])
                return carry2

            lax.fori_loop(0, BLK // 16, group, 0)

        def run_pipelined(idx_fn, gather_fn, compute_fn, store_fn=None):
            def wait_stores(S):
                for sc_ in store_fn(S, 0):
                    sc_.wait()

            @pl.when(wid < nblk)
            def _():
                fetch_block(idx_fn, gather_fn, sets[0], wid)

            def pair(h, carry):
                for p in (0, 1):
                    g = 2 * h + p
                    bb = wid + g * nw
                    bbn = bb + nw

                    @pl.when(bbn < nblk)
                    def _(q=1 - p, bbn=bbn, p=p, h=h):
                        if store_fn is not None:
                            if p == 0:
                                @pl.when(h >= 1)
                                def _(q=q):
                                    wait_stores(sets[q])
                            else:
                                wait_stores(sets[q])
                        fetch_block(idx_fn, gather_fn, sets[q], bbn)

                    @pl.when(bb < nblk)
                    def _(p=p, bb=bb):
                        wait_gathers(gather_fn, sets[p])
                        compute_fn(sets[p], bb)
                return carry

            lax.fori_loop(0, npair, pair, 0)
            if store_fn is not None:
                @pl.when(wid < nblk)
                def _():
                    wait_stores(sets[0])

                @pl.when(wid + nw < nblk)
                def _():
                    wait_stores(sets[1])

        run_pipelined(idx_copies_local, gather_copies_local, compute_local,
                      store_copies_local)
        run_pipelined(idx_copies_global, gather_copies_global, compute_global)
        pltpu.sync_copy(bins, out_hbm.at[wid])

    buf_set = [
        pltpu.VMEM((BLK, SDIM), F32),
        pltpu.VMEM((BLK, SDIM), F32),
        pltpu.VMEM((BLK,), I32),
        pltpu.VMEM((BLK,), I32),
        pltpu.VMEM((BLK,), I32),
        pltpu.VMEM((BLK,), F32),
        pltpu.VMEM((BLK,), F32),
        pltpu.VMEM((BLK,), F32),
        pltpu.VMEM((BLK,), F32),
        pltpu.VMEM((BLK,), F32),
        pltpu.VMEM((BLK,), F32),
        pltpu.VMEM((BLK,), F32),
        pltpu.VMEM((BLK,), F32),
        pltpu.VMEM((BLK,), I32),
    ]
    return pl.kernel(
        body,
        mesh=plsc.VectorSubcoreMesh(core_axis_name="c", subcore_axis_name="s"),
        compiler_params=pltpu.CompilerParams(use_tc_tiling_on_sc=False),
        out_type=[
            jax.ShapeDtypeStruct((nw, BINSZ), F32),
            jax.ShapeDtypeStruct((n_edges, SDIM), F32),
            jax.ShapeDtypeStruct((n_edges,), F32),
        ],
        scratch_types=buf_set + buf_set + [
            pltpu.VMEM((512, 64), F32),
            pltpu.VMEM((144,), F32),
            pltpu.VMEM((BINSZ,), F32),
            pltpu.SemaphoreType.DMA,
            pltpu.SemaphoreType.DMA,
            pltpu.SemaphoreType.DMA,
            pltpu.SemaphoreType.DMA,
            pltpu.SemaphoreType.DMA,
            pltpu.SemaphoreType.DMA,
        ],
    )


def kernel(x, pos, edge_index_local, edge_index_global, edge_attr_local,
           edge_attr_global, batch, atom_emb, bond_emb, W1, b1, W2, b2, w_g):
    n = x.shape[0]
    n_edges = edge_index_local.shape[1]
    x = x.astype(I32)
    pos = pos.astype(F32)
    srcl = edge_index_local[0].astype(I32)
    dstl = edge_index_local[1].astype(I32)
    srcg = edge_index_global[0].astype(I32)
    dstg = edge_index_global[1].astype(I32)
    eal = edge_attr_local.astype(I32)
    bidx = eal[:, 0] * 64 + eal[:, 1] * 8 + eal[:, 2]
    bat = batch.astype(I32)

    emb = atom_emb.astype(F32).reshape(9 * 64, 64)
    W1 = W1.astype(F32)
    pa, pb, av = _node_stage(x, emb, W1[:64], W1[64:128],
                             w_g.astype(F32).reshape(64, 1))
    bond_all = _bond_stage(bond_emb.astype(F32), W1[128:144],
                           b1.astype(F32).reshape(1, 64))
    consts = jnp.concatenate([W2[:, 0].astype(F32), W1[144],
                              b2.astype(F32), jnp.zeros((15,), F32)])

    try:
        info = plsc.get_sparse_core_info()
        nc, ns = info.num_cores, info.num_subcores
    except Exception:
        nc, ns = 2, 16
    sc_main = _make_sc_main(n_edges, nc, ns)
    partials, u, bate = sc_main(
        pa, pb, bond_all, pos[:, 0], pos[:, 1], pos[:, 2],
        bat, bat.astype(F32), av.reshape(n), srcl, dstl, bidx, srcg, dstg,
        consts)
    tc_energy = _mlp_stage(u, bate.reshape(n_edges // 128, 128),
                           W2.astype(F32).reshape(64, 1),
                           b2.astype(F32).reshape(1, 1))
    out = _reduce_stage(partials, tc_energy.reshape(1, NUM_GRAPHS))
    return out.reshape(NUM_GRAPHS)
